# baseline (speedup 1.0000x reference)
import jax
import jax.numpy as jnp
from jax import lax
from jax.experimental import pallas as pl
from jax.experimental.pallas import tpu as pltpu
from jax.experimental.pallas import tpu_sc as plsc

N_ROWS = 1000000
B_U = 4096
B_I = 4096
D = 32
PK = 128 // D

_info = plsc.get_sparse_core_info()
_NC = _info.num_cores
_NS = _info.num_subcores
_NW = _NC * _NS
_UB = B_U // _NW
_IB = B_I // _NW

_mesh = plsc.VectorSubcoreMesh(core_axis_name="c", subcore_axis_name="s")


def _gather_body(users_hbm, items_hbm, uf2, if2, u_out, v_out,
                 uidx_v, vidx_v, uq_v, vq_v, ug, vg, urows, vrows,
                 usem, vsem):
    wid = lax.axis_index("s") * _NC + lax.axis_index("c")
    ubase = wid * _UB
    ibase = wid * _IB
    pltpu.sync_copy(users_hbm.at[pl.ds(ubase, _UB)], uidx_v)
    pltpu.sync_copy(items_hbm.at[pl.ds(ibase, _IB)], vidx_v)

    for b in range(_UB // 16):
        w = uidx_v[pl.ds(16 * b, 16)]
        uq_v[pl.ds(16 * b, 16)] = lax.shift_right_logical(w, 2)
    for b in range(_IB // 16):
        w = vidx_v[pl.ds(16 * b, 16)]
        vq_v[pl.ds(16 * b, 16)] = lax.shift_right_logical(w, 2)

    cu = pltpu.async_copy(uf2.at[uq_v], ug, usem)
    cv = pltpu.async_copy(if2.at[vq_v], vg, vsem)
    cu.wait()
    cv.wait()

    def chunk(c, _):
        base = c * 16
        uw = uidx_v[pl.ds(base, 16)]
        uk = lax.mul(lax.bitwise_and(uw, PK - 1), D)
        vw = vidx_v[pl.ds(base, 16)]
        vk = lax.mul(lax.bitwise_and(vw, PK - 1), D)
        for j in range(16):
            r = base + j
            urows[r, pl.ds(0, 16)] = ug[r, pl.ds(uk[j], 16)]
            urows[r, pl.ds(16, 16)] = ug[r, pl.ds(uk[j] + 16, 16)]
            vrows[r, pl.ds(0, 16)] = vg[r, pl.ds(vk[j], 16)]
            vrows[r, pl.ds(16, 16)] = vg[r, pl.ds(vk[j] + 16, 16)]
        return 0

    lax.fori_loop(0, _UB // 16, chunk, 0)
    pltpu.sync_copy(urows, u_out.at[pl.ds(ubase, _UB)])
    pltpu.sync_copy(vrows, v_out.at[pl.ds(ibase, _IB)])


_gather = pl.kernel(
    _gather_body,
    mesh=_mesh,
    out_type=[
        jax.ShapeDtypeStruct((B_U, D), jnp.float32),
        jax.ShapeDtypeStruct((B_I, D), jnp.float32),
    ],
    scratch_types=[
        pltpu.VMEM((_UB,), jnp.int32),
        pltpu.VMEM((_IB,), jnp.int32),
        pltpu.VMEM((_UB,), jnp.int32),
        pltpu.VMEM((_IB,), jnp.int32),
        pltpu.VMEM((_UB, 128), jnp.float32),
        pltpu.VMEM((_IB, 128), jnp.float32),
        pltpu.VMEM((_UB, D), jnp.float32),
        pltpu.VMEM((_IB, D), jnp.float32),
        pltpu.SemaphoreType.DMA,
        pltpu.SemaphoreType.DMA,
    ],
)

_TM = 256


def _mm_body(u_ref, v_ref, o_ref):
    o_ref[...] = lax.dot_general(
        u_ref[...], v_ref[...],
        dimension_numbers=(((1,), (1,)), ((), ())),
        preferred_element_type=jnp.float32)


_matmul = pl.pallas_call(
    _mm_body,
    grid=(B_U // _TM,),
    in_specs=[
        pl.BlockSpec((_TM, D), lambda i: (i, 0)),
        pl.BlockSpec((B_I, D), lambda i: (0, 0)),
    ],
    out_specs=pl.BlockSpec((_TM, B_I), lambda i: (i, 0)),
    out_shape=jax.ShapeDtypeStruct((B_U, B_I), jnp.float32),
)


def kernel(users, items, user_factors, item_factors):
    uf2 = user_factors.reshape(N_ROWS // PK, 128)
    if2 = item_factors.reshape(N_ROWS // PK, 128)
    u, v = _gather(users, items, uf2, if2)
    return _matmul(u, v)


# speedup vs baseline: 1.4913x; 1.4913x over previous
import jax
import jax.numpy as jnp
from jax import lax
from jax.experimental import pallas as pl
from jax.experimental.pallas import tpu as pltpu
from jax.experimental.pallas import tpu_sc as plsc

B_U = 4096
B_I = 4096
D = 32

_info = plsc.get_sparse_core_info()
_NC = _info.num_cores
_NS = _info.num_subcores
_NW = _NC * _NS
_UB = B_U // _NW
_IB = B_I // _NW

_mesh = plsc.VectorSubcoreMesh(core_axis_name="c", subcore_axis_name="s")

_NSEM = 8
_CH = 8
_LOOK = 4


def _gather_body(users_hbm, items_hbm, uf_hbm, if_hbm, u_out, v_out,
                 uidx_v, vidx_v, urows, vrows, *sems):
    wid = lax.axis_index("s") * _NC + lax.axis_index("c")
    ubase = wid * _UB
    ibase = wid * _IB
    pltpu.sync_copy(users_hbm.at[pl.ds(ubase, _UB)], uidx_v)
    pltpu.sync_copy(items_hbm.at[pl.ds(ibase, _IB)], vidx_v)

    def fire(base):
        uw = uidx_v[pl.ds(base, _CH)]
        vw = vidx_v[pl.ds(base, _CH)]
        for j in range(_CH):
            pltpu.make_async_copy(
                uf_hbm.at[pl.ds(uw[j], 1)], urows.at[pl.ds(base + j, 1)],
                sems[j % _NSEM]).start()
            pltpu.make_async_copy(
                if_hbm.at[pl.ds(vw[j], 1)], vrows.at[pl.ds(base + j, 1)],
                sems[j % _NSEM]).start()

    def drain(base):
        for j in range(_CH):
            pltpu.make_async_copy(
                uf_hbm.at[pl.ds(0, 1)], urows.at[pl.ds(base + j, 1)],
                sems[j % _NSEM]).wait()
            pltpu.make_async_copy(
                if_hbm.at[pl.ds(0, 1)], vrows.at[pl.ds(base + j, 1)],
                sems[j % _NSEM]).wait()

    for p in range(_LOOK):
        fire(p * _CH)

    def chunk(c, _):
        fire(c * _CH)
        drain((c - _LOOK) * _CH)
        return 0

    lax.fori_loop(_LOOK, _UB // _CH, chunk, 0)
    for p in range(_LOOK):
        drain(_UB - (_LOOK - p) * _CH)
    pltpu.sync_copy(urows, u_out.at[pl.ds(ubase, _UB)])
    pltpu.sync_copy(vrows, v_out.at[pl.ds(ibase, _IB)])


_gather = pl.kernel(
    _gather_body,
    mesh=_mesh,
    out_type=[
        jax.ShapeDtypeStruct((B_U, D), jnp.float32),
        jax.ShapeDtypeStruct((B_I, D), jnp.float32),
    ],
    scratch_types=[
        pltpu.VMEM((_UB,), jnp.int32),
        pltpu.VMEM((_IB,), jnp.int32),
        pltpu.VMEM((_UB, D), jnp.float32),
        pltpu.VMEM((_IB, D), jnp.float32),
    ] + [pltpu.SemaphoreType.DMA] * _NSEM,
)

_TM = 256


def _mm_body(u_ref, v_ref, o_ref):
    o_ref[...] = lax.dot_general(
        u_ref[...], v_ref[...],
        dimension_numbers=(((1,), (1,)), ((), ())),
        preferred_element_type=jnp.float32)


_matmul = pl.pallas_call(
    _mm_body,
    grid=(B_U // _TM,),
    in_specs=[
        pl.BlockSpec((_TM, D), lambda i: (i, 0)),
        pl.BlockSpec((B_I, D), lambda i: (0, 0)),
    ],
    out_specs=pl.BlockSpec((_TM, B_I), lambda i: (i, 0)),
    out_shape=jax.ShapeDtypeStruct((B_U, B_I), jnp.float32),
)


def kernel(users, items, user_factors, item_factors):
    u, v = _gather(users, items, user_factors, item_factors)
    return _matmul(u, v)
